# Initial kernel scaffold; baseline (speedup 1.0000x reference)
import jax
import jax.numpy as jnp
from jax.experimental import pallas as pl


def kernel(data, fresh_data):
    raise NotImplementedError("write your pallas kernel here")


# speedup vs baseline: 2.2448x; 2.2448x over previous
import functools

import jax
import jax.numpy as jnp
from jax import lax
from jax.experimental import pallas as pl
from jax.experimental.pallas import tpu as pltpu
from jax.experimental.pallas import tpu_sc as plsc

_NUM_STEPS = 50
_NUM_ENVS = 4096
_FEAT = 128
_NUM_WORKERS = 32
_EPW = _NUM_ENVS // _NUM_WORKERS


def _sc_body(data_hbm, fresh_hbm, out_hbm, buf):
    wid = lax.axis_index("s") * 2 + lax.axis_index("c")
    e0 = wid * _EPW

    pltpu.sync_copy(fresh_hbm.at[pl.ds(e0, _EPW), :], buf)
    pltpu.sync_copy(buf, out_hbm.at[pl.ds(e0, _EPW), pl.ds(0, _FEAT)])
    for s in range(1, _NUM_STEPS):
        pltpu.sync_copy(data_hbm.at[s - 1, pl.ds(e0, _EPW), :], buf)
        pltpu.sync_copy(buf, out_hbm.at[pl.ds(e0, _EPW), pl.ds(s * _FEAT, _FEAT)])


def kernel(data, fresh_data):
    mesh = plsc.VectorSubcoreMesh(core_axis_name="c", subcore_axis_name="s")
    run = pl.kernel(
        _sc_body,
        out_type=jax.ShapeDtypeStruct((_NUM_ENVS, _NUM_STEPS * _FEAT), jnp.float32),
        mesh=mesh,
        scratch_types=[
            pltpu.VMEM((_EPW, _FEAT), jnp.float32),
        ],
    )
    return run(data, fresh_data)


# speedup vs baseline: 3.1124x; 1.3865x over previous
import functools

import jax
import jax.numpy as jnp
from jax import lax
from jax.experimental import pallas as pl
from jax.experimental.pallas import tpu as pltpu
from jax.experimental.pallas import tpu_sc as plsc

_NUM_STEPS = 50
_NUM_ENVS = 4096
_FEAT = 128
_NUM_WORKERS = 32
_EPW = _NUM_ENVS // _NUM_WORKERS


_NBUF = 4
_DEPTH = 2


def _sc_body(data_hbm, fresh_hbm, out_hbm, *scratch):
    bufs = scratch[:_NBUF]
    isems = scratch[_NBUF:2 * _NBUF]
    osems = scratch[2 * _NBUF:]
    wid = lax.axis_index("s") * 2 + lax.axis_index("c")
    e0 = wid * _EPW

    def src(s):
        if s == 0:
            return fresh_hbm.at[pl.ds(e0, _EPW), :]
        return data_hbm.at[s - 1, pl.ds(e0, _EPW), :]

    def dst(s):
        return out_hbm.at[pl.ds(e0, _EPW), pl.ds(s * _FEAT, _FEAT)]

    inc = [None] * _NUM_STEPS
    outc = [None] * _NUM_STEPS
    for s in range(_DEPTH):
        inc[s] = pltpu.async_copy(src(s), bufs[s % _NBUF], isems[s % _NBUF])
    for s in range(_NUM_STEPS):
        b = s % _NBUF
        inc[s].wait()
        outc[s] = pltpu.async_copy(bufs[b], dst(s), osems[b])
        ns = s + _DEPTH
        if ns < _NUM_STEPS:
            if ns >= _NBUF:
                outc[ns - _NBUF].wait()
            inc[ns] = pltpu.async_copy(src(ns), bufs[ns % _NBUF], isems[ns % _NBUF])
    for s in range(_NUM_STEPS - _NBUF, _NUM_STEPS):
        outc[s].wait()


def kernel(data, fresh_data):
    mesh = plsc.VectorSubcoreMesh(core_axis_name="c", subcore_axis_name="s")
    run = pl.kernel(
        _sc_body,
        out_type=jax.ShapeDtypeStruct((_NUM_ENVS, _NUM_STEPS * _FEAT), jnp.float32),
        mesh=mesh,
        scratch_types=(
            [pltpu.VMEM((_EPW, _FEAT), jnp.float32) for _ in range(_NBUF)]
            + [pltpu.SemaphoreType.DMA for _ in range(2 * _NBUF)]
        ),
    )
    return run(data, fresh_data)


# speedup vs baseline: 3.1335x; 1.0068x over previous
import functools

import jax
import jax.numpy as jnp
from jax import lax
from jax.experimental import pallas as pl
from jax.experimental.pallas import tpu as pltpu
from jax.experimental.pallas import tpu_sc as plsc

_NUM_STEPS = 50
_NUM_ENVS = 4096
_FEAT = 128
_NUM_WORKERS = 32
_EPW = _NUM_ENVS // _NUM_WORKERS


_NBUF = 6
_DEPTH = 3


def _sc_body(data_hbm, fresh_hbm, out_hbm, *scratch):
    bufs = scratch[:_NBUF]
    isems = scratch[_NBUF:2 * _NBUF]
    osems = scratch[2 * _NBUF:]
    wid = lax.axis_index("s") * 2 + lax.axis_index("c")
    e0 = wid * _EPW

    def src(s):
        if s == 0:
            return fresh_hbm.at[pl.ds(e0, _EPW), :]
        return data_hbm.at[s - 1, pl.ds(e0, _EPW), :]

    def dst(s):
        return out_hbm.at[pl.ds(e0, _EPW), pl.ds(s * _FEAT, _FEAT)]

    inc = [None] * _NUM_STEPS
    outc = [None] * _NUM_STEPS
    for s in range(_DEPTH):
        inc[s] = pltpu.async_copy(src(s), bufs[s % _NBUF], isems[s % _NBUF])
    for s in range(_NUM_STEPS):
        b = s % _NBUF
        inc[s].wait()
        outc[s] = pltpu.async_copy(bufs[b], dst(s), osems[b])
        ns = s + _DEPTH
        if ns < _NUM_STEPS:
            if ns >= _NBUF:
                outc[ns - _NBUF].wait()
            inc[ns] = pltpu.async_copy(src(ns), bufs[ns % _NBUF], isems[ns % _NBUF])
    for s in range(_NUM_STEPS - _NBUF, _NUM_STEPS):
        outc[s].wait()


def kernel(data, fresh_data):
    mesh = plsc.VectorSubcoreMesh(core_axis_name="c", subcore_axis_name="s")
    run = pl.kernel(
        _sc_body,
        out_type=jax.ShapeDtypeStruct((_NUM_ENVS, _NUM_STEPS * _FEAT), jnp.float32),
        mesh=mesh,
        scratch_types=(
            [pltpu.VMEM((_EPW, _FEAT), jnp.float32) for _ in range(_NBUF)]
            + [pltpu.SemaphoreType.DMA for _ in range(2 * _NBUF)]
        ),
    )
    return run(data, fresh_data)
